# Initial kernel scaffold; baseline (speedup 1.0000x reference)
import functools

import jax
import jax.numpy as jnp
from jax import lax
from jax.experimental import pallas as pl
from jax.experimental.pallas import tpu as pltpu
from jax.experimental.pallas import tpu_sc as plsc

N = 10000
E = 320000
H = 128
VOCAB = 10000

L = 16
NT = 16
NP = 10240
SL = NP // NT
EW = E // NT


def _rsqrt16(x):
    xi = plsc.bitcast(x, jnp.int32)
    yi = jnp.int32(0x5F3759DF) - lax.shift_right_arithmetic(xi, 1)
    y = plsc.bitcast(yi, jnp.float32)
    for _ in range(3):
        y = y * (1.5 - 0.5 * x * y * y)
    return jnp.where(x > 0.0, y, 0.0)


def _zero_vmem(ref, n):
    zeros = jnp.zeros((L,), jnp.float32)

    def body(i, _):
        ref[pl.ds(i * L, L)] = zeros
        return 0

    lax.fori_loop(0, n // L, body, 0)


def _sc_body(src_hbm, dst_hbm, feats_hbm, c_out, w_out,
             srcv, dstv, acc1, acc2, full, red, nsl, ndl, tsl, fsl,
             shp1, shp2, shf):
    core = lax.axis_index("c")
    sid = lax.axis_index("s")

    @pl.when(core == 0)
    def _():
        wid = sid
        base_e = wid * EW
        base_n = wid * SL
        ones = jnp.ones((L,), jnp.float32)

        pltpu.sync_copy(src_hbm.at[pl.ds(base_e, EW)], srcv)
        pltpu.sync_copy(dst_hbm.at[pl.ds(base_e, EW)], dstv)

        _zero_vmem(acc1, NP)
        _zero_vmem(acc2, NP)

        def deg_body(i, _):
            si = srcv[pl.ds(i * L, L)]
            di = dstv[pl.ds(i * L, L)]
            plsc.addupdate_scatter(acc1, [si], ones)
            plsc.addupdate_scatter(acc2, [di], ones)
            return 0

        lax.fori_loop(0, EW // L, deg_body, 0)

        pltpu.sync_copy(acc1, shp1.at[wid])
        pltpu.sync_copy(acc2, shp2.at[wid])
        plsc.subcore_barrier()

        for t in range(NT):
            pltpu.sync_copy(shp1.at[t, pl.ds(base_n, SL)], red.at[t])

        def ns_body(j, _):
            s = red[0, pl.ds(j * L, L)]
            for t in range(1, NT):
                s = s + red[t, pl.ds(j * L, L)]
            nsl[pl.ds(j * L, L)] = _rsqrt16(s)
            return 0

        lax.fori_loop(0, SL // L, ns_body, 0)

        for t in range(NT):
            pltpu.sync_copy(shp2.at[t, pl.ds(base_n, SL)], red.at[t])

        def nd_body(j, _):
            s = red[0, pl.ds(j * L, L)]
            for t in range(1, NT):
                s = s + red[t, pl.ds(j * L, L)]
            ndl[pl.ds(j * L, L)] = _rsqrt16(s)
            return 0

        lax.fori_loop(0, SL // L, nd_body, 0)

        pltpu.sync_copy(ndl, shf.at[pl.ds(base_n, SL)])
        plsc.subcore_barrier()
        pltpu.sync_copy(shf, full)

        _zero_vmem(acc1, NP)

        def w_body(i, _):
            si = srcv[pl.ds(i * L, L)]
            di = dstv[pl.ds(i * L, L)]
            g = plsc.load_gather(full, [di])
            plsc.addupdate_scatter(acc1, [si], g)
            return 0

        lax.fori_loop(0, EW // L, w_body, 0)

        pltpu.sync_copy(acc1, shp1.at[wid])
        plsc.subcore_barrier()

        for t in range(NT):
            pltpu.sync_copy(shp1.at[t, pl.ds(base_n, SL)], red.at[t])

        def wt_body(j, _):
            s = red[0, pl.ds(j * L, L)]
            for t in range(1, NT):
                s = s + red[t, pl.ds(j * L, L)]
            wv = nsl[pl.ds(j * L, L)] * s
            tsl[pl.ds(j * L, L)] = wv
            ndl[pl.ds(j * L, L)] = ndl[pl.ds(j * L, L)] * wv
            return 0

        lax.fori_loop(0, SL // L, wt_body, 0)

        pltpu.sync_copy(tsl, w_out.at[pl.ds(base_n, SL)])
        pltpu.sync_copy(ndl, shf.at[pl.ds(base_n, SL)])
        plsc.subcore_barrier()
        pltpu.sync_copy(shf, full)

        _zero_vmem(acc2, NP)

        def u_body(i, _):
            si = srcv[pl.ds(i * L, L)]
            di = dstv[pl.ds(i * L, L)]
            g = plsc.load_gather(full, [di])
            plsc.addupdate_scatter(acc2, [si], g)
            return 0

        lax.fori_loop(0, EW // L, u_body, 0)

        pltpu.sync_copy(acc2, shp2.at[wid])
        plsc.subcore_barrier()

        for t in range(NT):
            pltpu.sync_copy(shp2.at[t, pl.ds(base_n, SL)], red.at[t])

        def u_red_body(j, _):
            s = red[0, pl.ds(j * L, L)]
            for t in range(1, NT):
                s = s + red[t, pl.ds(j * L, L)]
            ndl[pl.ds(j * L, L)] = nsl[pl.ds(j * L, L)] * s
            return 0

        lax.fori_loop(0, SL // L, u_red_body, 0)

        pltpu.sync_copy(feats_hbm.at[pl.ds(base_n, SL)], fsl)
        _zero_vmem(acc1, NP)

        def c_body(j, _):
            fv = fsl[pl.ds(j * L, L)]
            uv = ndl[pl.ds(j * L, L)]
            plsc.addupdate_scatter(acc1, [fv], uv)
            return 0

        lax.fori_loop(0, SL // L, c_body, 0)

        pltpu.sync_copy(acc1, shp1.at[wid])
        plsc.subcore_barrier()

        for t in range(NT):
            pltpu.sync_copy(shp1.at[t, pl.ds(base_n, SL)], red.at[t])

        def c_red_body(j, _):
            s = red[0, pl.ds(j * L, L)]
            for t in range(1, NT):
                s = s + red[t, pl.ds(j * L, L)]
            tsl[pl.ds(j * L, L)] = s
            return 0

        lax.fori_loop(0, SL // L, c_red_body, 0)

        pltpu.sync_copy(tsl, c_out.at[pl.ds(base_n, SL)])


_sc_call = functools.partial(
    pl.kernel,
    out_type=[
        jax.ShapeDtypeStruct((NP,), jnp.float32),
        jax.ShapeDtypeStruct((NP,), jnp.float32),
    ],
    mesh=plsc.VectorSubcoreMesh(core_axis_name="c", subcore_axis_name="s"),
    scratch_types=[
        pltpu.VMEM((EW,), jnp.int32),
        pltpu.VMEM((EW,), jnp.int32),
        pltpu.VMEM((NP,), jnp.float32),
        pltpu.VMEM((NP,), jnp.float32),
        pltpu.VMEM((NP,), jnp.float32),
        pltpu.VMEM((NT, SL), jnp.float32),
        pltpu.VMEM((SL,), jnp.float32),
        pltpu.VMEM((SL,), jnp.float32),
        pltpu.VMEM((SL,), jnp.float32),
        pltpu.VMEM((SL,), jnp.int32),
        pltpu.VMEM_SHARED((NT, NP), jnp.float32),
        pltpu.VMEM_SHARED((NT, NP), jnp.float32),
        pltpu.VMEM_SHARED((NP,), jnp.float32),
    ],
)(_sc_body)


def _tc_body(c_ref, w_ref, emb_ref, w0_ref, b0_ref, w1_ref, b1_ref,
             wreg_ref, o_ref):
    c = c_ref[...]
    z = jnp.dot(c, emb_ref[...], preferred_element_type=jnp.float32)
    sw = jnp.sum(w_ref[...])
    r1 = jnp.dot(z, w0_ref[...], preferred_element_type=jnp.float32)
    r1 = r1 + sw * b0_ref[...]
    r2 = jnp.dot(r1, w1_ref[...], preferred_element_type=jnp.float32)
    r2 = r2 + float(N) * b1_ref[...]
    o_ref[...] = jnp.sum(r2 * wreg_ref[...], axis=1, keepdims=True)


def kernel(feats, edge_index, emb, W0, b0, W1, b1, Wreg):
    src = edge_index[0].astype(jnp.int32)
    dst = edge_index[1].astype(jnp.int32)
    feats_pad = jnp.pad(feats.astype(jnp.int32), (0, NP - N))
    emb_pad = jnp.pad(emb, ((0, NP - VOCAB), (0, 0)))

    c, w = _sc_call(src, dst, feats_pad)

    out = pl.pallas_call(
        _tc_body,
        out_shape=jax.ShapeDtypeStruct((1, 1), jnp.float32),
    )(c.reshape(1, NP), w.reshape(1, NP), emb_pad,
      W0, b0.reshape(1, H), W1, b1.reshape(1, H), Wreg)
    return out


# speedup vs baseline: 28.9538x; 28.9538x over previous
import functools

import jax
import jax.numpy as jnp
from jax import lax
from jax.experimental import pallas as pl
from jax.experimental.pallas import tpu as pltpu
from jax.experimental.pallas import tpu_sc as plsc

N = 10000
E = 320000
H = 128
VOCAB = 10000

L = 16
NT = 16
NP = 10240
SL = NP // NT
EW = E // NT


def _rsqrt16(x):
    xi = plsc.bitcast(x, jnp.int32)
    yi = jnp.int32(0x5F3759DF) - lax.shift_right_arithmetic(xi, 1)
    y = plsc.bitcast(yi, jnp.float32)
    for _ in range(3):
        y = y * (1.5 - 0.5 * x * y * y)
    return jnp.where(x > 0.0, y, 0.0)


def _zero_vmem(ref, n):
    zeros = jnp.zeros((L,), jnp.float32)

    def body(i, _):
        ref[pl.ds(i * L, L)] = zeros
        return 0

    lax.fori_loop(0, n // L, body, 0)


def _sc_body(src_hbm, dst_hbm, feats_hbm, c_out, w_out,
             srcv, dstv, acc1, acc2, full, red, nsl, ndl, tsl, fsl,
             shp1, shp2, shf):
    core = lax.axis_index("c")
    sid = lax.axis_index("s")

    @pl.when(core == 0)
    def _():
        wid = sid
        base_e = wid * EW
        base_n = wid * SL
        ones = jnp.ones((L,), jnp.float32)

        pltpu.sync_copy(src_hbm.at[pl.ds(base_e, EW)], srcv)
        pltpu.sync_copy(dst_hbm.at[pl.ds(base_e, EW)], dstv)

        _zero_vmem(acc1, NP)
        _zero_vmem(acc2, NP)

        def deg_body(i, _):
            si = srcv[pl.ds(i * L, L)]
            di = dstv[pl.ds(i * L, L)]
            plsc.addupdate_scatter(acc1, [si], ones)
            plsc.addupdate_scatter(acc2, [di], ones)
            return 0

        lax.fori_loop(0, EW // L, deg_body, 0)

        pltpu.sync_copy(acc1, shp1.at[wid])
        pltpu.sync_copy(acc2, shp2.at[wid])
        plsc.subcore_barrier()

        for t in range(NT):
            pltpu.sync_copy(shp1.at[t, pl.ds(base_n, SL)], red.at[t])

        def ns_body(j, _):
            s = red[0, pl.ds(j * L, L)]
            for t in range(1, NT):
                s = s + red[t, pl.ds(j * L, L)]
            nsl[pl.ds(j * L, L)] = _rsqrt16(s)
            return 0

        lax.fori_loop(0, SL // L, ns_body, 0)

        for t in range(NT):
            pltpu.sync_copy(shp2.at[t, pl.ds(base_n, SL)], red.at[t])

        def nd_body(j, _):
            s = red[0, pl.ds(j * L, L)]
            for t in range(1, NT):
                s = s + red[t, pl.ds(j * L, L)]
            ndl[pl.ds(j * L, L)] = _rsqrt16(s)
            return 0

        lax.fori_loop(0, SL // L, nd_body, 0)

        pltpu.sync_copy(ndl, shf.at[pl.ds(base_n, SL)])
        plsc.subcore_barrier()
        pltpu.sync_copy(shf, full)

        _zero_vmem(acc1, NP)

        def w_body(i, _):
            si = srcv[pl.ds(i * L, L)]
            di = dstv[pl.ds(i * L, L)]
            g = plsc.load_gather(full, [di])
            plsc.addupdate_scatter(acc1, [si], g)
            return 0

        lax.fori_loop(0, EW // L, w_body, 0)

        pltpu.sync_copy(acc1, shp1.at[wid])
        plsc.subcore_barrier()

        for t in range(NT):
            pltpu.sync_copy(shp1.at[t, pl.ds(base_n, SL)], red.at[t])

        def wt_body(j, _):
            s = red[0, pl.ds(j * L, L)]
            for t in range(1, NT):
                s = s + red[t, pl.ds(j * L, L)]
            wv = nsl[pl.ds(j * L, L)] * s
            tsl[pl.ds(j * L, L)] = wv
            ndl[pl.ds(j * L, L)] = ndl[pl.ds(j * L, L)] * wv
            return 0

        lax.fori_loop(0, SL // L, wt_body, 0)

        pltpu.sync_copy(tsl, w_out.at[pl.ds(base_n, SL)])
        pltpu.sync_copy(ndl, shf.at[pl.ds(base_n, SL)])
        plsc.subcore_barrier()
        pltpu.sync_copy(shf, full)

        _zero_vmem(acc2, NP)

        def u_body(i, _):
            si = srcv[pl.ds(i * L, L)]
            di = dstv[pl.ds(i * L, L)]
            g = plsc.load_gather(full, [di])
            plsc.addupdate_scatter(acc2, [si], g)
            return 0

        lax.fori_loop(0, EW // L, u_body, 0)

        pltpu.sync_copy(acc2, shp2.at[wid])
        plsc.subcore_barrier()

        for t in range(NT):
            pltpu.sync_copy(shp2.at[t, pl.ds(base_n, SL)], red.at[t])

        def u_red_body(j, _):
            s = red[0, pl.ds(j * L, L)]
            for t in range(1, NT):
                s = s + red[t, pl.ds(j * L, L)]
            ndl[pl.ds(j * L, L)] = nsl[pl.ds(j * L, L)] * s
            return 0

        lax.fori_loop(0, SL // L, u_red_body, 0)

        pltpu.sync_copy(feats_hbm.at[pl.ds(base_n, SL)], fsl)
        _zero_vmem(acc1, NP)

        def c_body(j, _):
            fv = fsl[pl.ds(j * L, L)]
            uv = ndl[pl.ds(j * L, L)]
            plsc.addupdate_scatter(acc1, [fv], uv)
            return 0

        lax.fori_loop(0, SL // L, c_body, 0)

        pltpu.sync_copy(acc1, shp1.at[wid])
        plsc.subcore_barrier()

        for t in range(NT):
            pltpu.sync_copy(shp1.at[t, pl.ds(base_n, SL)], red.at[t])

        def c_red_body(j, _):
            s = red[0, pl.ds(j * L, L)]
            for t in range(1, NT):
                s = s + red[t, pl.ds(j * L, L)]
            tsl[pl.ds(j * L, L)] = s
            return 0

        lax.fori_loop(0, SL // L, c_red_body, 0)

        pltpu.sync_copy(tsl, c_out.at[pl.ds(base_n, SL)])


_sc_call = functools.partial(
    pl.kernel,
    out_type=[
        jax.ShapeDtypeStruct((NP,), jnp.float32),
        jax.ShapeDtypeStruct((NP,), jnp.float32),
    ],
    mesh=plsc.VectorSubcoreMesh(core_axis_name="c", subcore_axis_name="s"),
    compiler_params=pltpu.CompilerParams(needs_layout_passes=False),
    scratch_types=[
        pltpu.VMEM((EW,), jnp.int32),
        pltpu.VMEM((EW,), jnp.int32),
        pltpu.VMEM((NP,), jnp.float32),
        pltpu.VMEM((NP,), jnp.float32),
        pltpu.VMEM((NP,), jnp.float32),
        pltpu.VMEM((NT, SL), jnp.float32),
        pltpu.VMEM((SL,), jnp.float32),
        pltpu.VMEM((SL,), jnp.float32),
        pltpu.VMEM((SL,), jnp.float32),
        pltpu.VMEM((SL,), jnp.int32),
        pltpu.VMEM_SHARED((NT, NP), jnp.float32),
        pltpu.VMEM_SHARED((NT, NP), jnp.float32),
        pltpu.VMEM_SHARED((NP,), jnp.float32),
    ],
)(_sc_body)


def _tc_body(c_ref, w_ref, emb_ref, w0_ref, b0_ref, w1_ref, b1_ref,
             wreg_ref, o_ref):
    ct = c_ref[...].reshape(NP, 1)
    z = jnp.sum(ct * emb_ref[...], axis=0, keepdims=True)
    sw = jnp.sum(w_ref[...])
    r1 = jnp.sum(z.reshape(H, 1) * w0_ref[...], axis=0, keepdims=True)
    r1 = r1 + sw * b0_ref[...]
    r2 = jnp.sum(r1.reshape(H, 1) * w1_ref[...], axis=0, keepdims=True)
    r2 = r2 + float(N) * b1_ref[...]
    o_ref[...] = jnp.sum(r2 * wreg_ref[...], axis=1, keepdims=True)


def kernel(feats, edge_index, emb, W0, b0, W1, b1, Wreg):
    src = edge_index[0].astype(jnp.int32)
    dst = edge_index[1].astype(jnp.int32)
    feats_pad = jnp.pad(feats.astype(jnp.int32), (0, NP - N))
    emb_pad = jnp.pad(emb, ((0, NP - VOCAB), (0, 0)))

    c, w = _sc_call(src, dst, feats_pad)

    out = pl.pallas_call(
        _tc_body,
        out_shape=jax.ShapeDtypeStruct((1, 1), jnp.float32),
    )(c.reshape(1, NP), w.reshape(1, NP), emb_pad,
      W0, b0.reshape(1, H), W1, b1.reshape(1, H), Wreg)
    return out


# speedup vs baseline: 33.8878x; 1.1704x over previous
import functools

import jax
import jax.numpy as jnp
from jax import lax
from jax.experimental import pallas as pl
from jax.experimental.pallas import tpu as pltpu
from jax.experimental.pallas import tpu_sc as plsc

N = 10000
E = 320000
H = 128
VOCAB = 10000

L = 16
NT = 16
NP = 10240
SL = NP // NT
EW = E // NT
FSL = N - 15 * SL


def _rsqrt16(x):
    xi = plsc.bitcast(x, jnp.int32)
    yi = jnp.int32(0x5F3759DF) - lax.shift_right_arithmetic(xi, 1)
    y = plsc.bitcast(yi, jnp.float32)
    for _ in range(3):
        y = y * (1.5 - 0.5 * x * y * y)
    return jnp.where(x > 0.0, y, 0.0)


def _zero_vmem(ref, n):
    zeros = jnp.zeros((L,), jnp.float32)

    def body(i, _):
        ref[pl.ds(i * L, L)] = zeros
        return 0

    lax.fori_loop(0, n // L, body, 0, unroll=8)


def _reduce16(red, out_slice_fn):
    def body(j, _):
        s = red[0, pl.ds(j * L, L)]
        for t in range(1, NT):
            s = s + red[t, pl.ds(j * L, L)]
        out_slice_fn(j, s)
        return 0

    lax.fori_loop(0, SL // L, body, 0)


def _sc_body(src_hbm, dst_hbm, feats_hbm, c_out, w_out,
             srcv, dstv, acc1, acc2, full, red, nsl, ndl, tsl, fsl,
             shp1, shp2, shf, sem):
    core = lax.axis_index("c")
    sid = lax.axis_index("s")

    @pl.when(core == 0)
    def _():
        wid = sid
        base_e = wid * EW
        base_n = wid * SL
        ones = jnp.ones((L,), jnp.float32)

        def pull_red(shp):
            cps = [pltpu.async_copy(shp.at[t, pl.ds(base_n, SL)], red.at[t],
                                    sem) for t in range(NT)]
            for cp in cps:
                cp.wait()

        with jax.named_scope("load_edges"):
            ld_s = pltpu.async_copy(src_hbm.at[pl.ds(base_e, EW)], srcv, sem)
            ld_d = pltpu.async_copy(dst_hbm.at[pl.ds(base_e, EW)], dstv, sem)

        with jax.named_scope("zero_deg"):
            _zero_vmem(acc1, NP)
            _zero_vmem(acc2, NP)
        ld_s.wait()
        ld_d.wait()

        with jax.named_scope("deg_pass"):
            def deg_body(i, _):
                si = srcv[pl.ds(i * L, L)]
                di = dstv[pl.ds(i * L, L)]
                plsc.addupdate_scatter(acc1, [si], ones)
                plsc.addupdate_scatter(acc2, [di], ones)
                return 0

            lax.fori_loop(0, EW // L, deg_body, 0, unroll=4)

        with jax.named_scope("deg_reduce"):
            pltpu.sync_copy(acc1, shp1.at[wid])
            pltpu.sync_copy(acc2, shp2.at[wid])
            plsc.subcore_barrier()

            pull_red(shp1)
            _reduce16(red, lambda j, s: nsl.__setitem__(pl.ds(j * L, L),
                                                        _rsqrt16(s)))
            pull_red(shp2)
            _reduce16(red, lambda j, s: ndl.__setitem__(pl.ds(j * L, L),
                                                        _rsqrt16(s)))

            pltpu.sync_copy(ndl, shf.at[pl.ds(base_n, SL)])
            plsc.subcore_barrier()
            pltpu.sync_copy(shf, full)

        with jax.named_scope("zero_w"):
            _zero_vmem(acc1, NP)

        with jax.named_scope("w_pass"):
            def w_body(i, _):
                si = srcv[pl.ds(i * L, L)]
                di = dstv[pl.ds(i * L, L)]
                g = plsc.load_gather(full, [di])
                plsc.addupdate_scatter(acc1, [si], g)
                return 0

            lax.fori_loop(0, EW // L, w_body, 0, unroll=4)

        with jax.named_scope("w_reduce"):
            pltpu.sync_copy(acc1, shp1.at[wid])
            plsc.subcore_barrier()

            pull_red(shp1)

            def wt_out(j, s):
                wv = nsl[pl.ds(j * L, L)] * s
                tsl[pl.ds(j * L, L)] = wv
                ndl[pl.ds(j * L, L)] = ndl[pl.ds(j * L, L)] * wv

            _reduce16(red, wt_out)
            pltpu.sync_copy(tsl, w_out.at[pl.ds(base_n, SL)])
            pltpu.sync_copy(ndl, shf.at[pl.ds(base_n, SL)])
            plsc.subcore_barrier()
            pltpu.sync_copy(shf, full)

        with jax.named_scope("zero_u"):
            _zero_vmem(acc2, NP)

        with jax.named_scope("u_pass"):
            def u_body(i, _):
                si = srcv[pl.ds(i * L, L)]
                di = dstv[pl.ds(i * L, L)]
                g = plsc.load_gather(full, [di])
                plsc.addupdate_scatter(acc2, [si], g)
                return 0

            lax.fori_loop(0, EW // L, u_body, 0, unroll=4)

        with jax.named_scope("u_reduce"):
            pltpu.sync_copy(acc2, shp2.at[wid])
            plsc.subcore_barrier()

            pull_red(shp2)
            _reduce16(red, lambda j, s: ndl.__setitem__(
                pl.ds(j * L, L), nsl[pl.ds(j * L, L)] * s))

        with jax.named_scope("c_pass"):
            @pl.when(sid < NT - 1)
            def _():
                pltpu.sync_copy(feats_hbm.at[pl.ds(base_n, SL)], fsl)

            @pl.when(sid == NT - 1)
            def _():
                pltpu.sync_copy(feats_hbm.at[pl.ds((NT - 1) * SL, FSL)],
                                fsl.at[pl.ds(0, FSL)])
                izeros = jnp.zeros((L,), jnp.int32)

                def zb(i, _):
                    fsl[pl.ds(FSL + i * L, L)] = izeros
                    return 0

                lax.fori_loop(0, (SL - FSL) // L, zb, 0, unroll=8)

            _zero_vmem(acc1, NP)

            def c_body(j, _):
                fv = fsl[pl.ds(j * L, L)]
                uv = ndl[pl.ds(j * L, L)]
                plsc.addupdate_scatter(acc1, [fv], uv)
                return 0

            lax.fori_loop(0, SL // L, c_body, 0, unroll=4)

        with jax.named_scope("c_reduce"):
            pltpu.sync_copy(acc1, shp1.at[wid])
            plsc.subcore_barrier()

            pull_red(shp1)
            _reduce16(red, lambda j, s: tsl.__setitem__(pl.ds(j * L, L), s))
            pltpu.sync_copy(tsl, c_out.at[pl.ds(base_n, SL)])


_sc_call = functools.partial(
    pl.kernel,
    out_type=[
        jax.ShapeDtypeStruct((NP,), jnp.float32),
        jax.ShapeDtypeStruct((NP,), jnp.float32),
    ],
    mesh=plsc.VectorSubcoreMesh(core_axis_name="c", subcore_axis_name="s"),
    compiler_params=pltpu.CompilerParams(needs_layout_passes=False),
    scratch_types=[
        pltpu.VMEM((EW,), jnp.int32),
        pltpu.VMEM((EW,), jnp.int32),
        pltpu.VMEM((NP,), jnp.float32),
        pltpu.VMEM((NP,), jnp.float32),
        pltpu.VMEM((NP,), jnp.float32),
        pltpu.VMEM((NT, SL), jnp.float32),
        pltpu.VMEM((SL,), jnp.float32),
        pltpu.VMEM((SL,), jnp.float32),
        pltpu.VMEM((SL,), jnp.float32),
        pltpu.VMEM((SL,), jnp.int32),
        pltpu.VMEM_SHARED((NT, NP), jnp.float32),
        pltpu.VMEM_SHARED((NT, NP), jnp.float32),
        pltpu.VMEM_SHARED((NP,), jnp.float32),
        pltpu.SemaphoreType.DMA,
    ],
)(_sc_body)


def _tc_body(c_ref, w_ref, emb_ref, w0_ref, b0_ref, w1_ref, b1_ref,
             wreg_ref, o_ref):
    w0 = w0_ref[...].astype(jnp.bfloat16).astype(jnp.float32)
    w1 = w1_ref[...].astype(jnp.bfloat16).astype(jnp.float32)
    ct = c_ref[0:VOCAB, :]
    z = jnp.sum(ct * emb_ref[...], axis=0, keepdims=True)
    sw = jnp.sum(w_ref[...])
    r1 = jnp.sum(z.reshape(H, 1) * w0, axis=0, keepdims=True)
    r1 = r1 + sw * b0_ref[...]
    r2 = jnp.sum(r1.reshape(H, 1) * w1, axis=0, keepdims=True)
    r2 = r2 + float(N) * b1_ref[...]
    o_ref[...] = jnp.sum(r2 * wreg_ref[...], axis=1, keepdims=True)


def kernel(feats, edge_index, emb, W0, b0, W1, b1, Wreg):
    edges = edge_index.astype(jnp.int32)
    c, w = _sc_call(edges[0], edges[1], feats.astype(jnp.int32))

    out = pl.pallas_call(
        _tc_body,
        out_shape=jax.ShapeDtypeStruct((1, 1), jnp.float32),
    )(c.reshape(NP, 1), w.reshape(1, NP), emb,
      W0, b0.reshape(1, H), W1, b1.reshape(1, H), Wreg)
    return out


# speedup vs baseline: 48.7650x; 1.4390x over previous
import functools

import jax
import jax.numpy as jnp
from jax import lax
from jax.experimental import pallas as pl
from jax.experimental.pallas import tpu as pltpu
from jax.experimental.pallas import tpu_sc as plsc

N = 10000
E = 320000
H = 128
VOCAB = 10000

L = 16
NT = 16
NP = 10240
SL = NP // NT
EC = 19968
ECL = E - (NT - 1) * EC
FSL = N - 15 * SL


def _rsqrt16(x):
    xi = plsc.bitcast(x, jnp.int32)
    yi = jnp.int32(0x5F3759DF) - lax.shift_right_arithmetic(xi, 1)
    y = plsc.bitcast(yi, jnp.float32)
    for _ in range(3):
        y = y * (1.5 - 0.5 * x * y * y)
    return jnp.where(x > 0.0, y, 0.0)


def _zero_vmem(ref, n):
    zeros = jnp.zeros((L,), jnp.float32)

    @plsc.parallel_loop(0, n // L, unroll=8)
    def _(i):
        ref[pl.ds(i * L, L)] = zeros


def _reduce16(red, out_slice_fn):
    def body(j, _):
        s = red[0, pl.ds(j * L, L)]
        for t in range(1, NT):
            s = s + red[t, pl.ds(j * L, L)]
        out_slice_fn(j, s)
        return 0

    lax.fori_loop(0, SL // L, body, 0)


def _sc_body(edges_hbm, feats_hbm, c_out, w_out,
             edgv, acc1, acc2, full, red, nsl, ndl, tsl, fsl,
             shp1, shp2, shf, sem):
    core = lax.axis_index("c")
    sid = lax.axis_index("s")

    @pl.when(core == 0)
    def _():
        wid = sid
        base_e = wid * EC
        base_n = wid * SL
        ones = jnp.ones((L,), jnp.float32)
        last = sid == NT - 1

        def pull_red(shp):
            cps = [pltpu.async_copy(shp.at[t, pl.ds(base_n, SL)], red.at[t],
                                    sem) for t in range(NT)]
            for cp in cps:
                cp.wait()

        def deg_pass(nv):
            @plsc.parallel_loop(0, nv, unroll=4)
            def _(i):
                si = edgv[0, pl.ds(i * L, L)]
                di = edgv[1, pl.ds(i * L, L)]
                plsc.addupdate_scatter(acc1, [si], ones)
                plsc.addupdate_scatter(acc2, [di], ones)

        def gs_pass(nv, acc):
            @plsc.parallel_loop(0, nv, unroll=4)
            def _(i):
                si = edgv[0, pl.ds(i * L, L)]
                di = edgv[1, pl.ds(i * L, L)]
                g = plsc.load_gather(full, [di])
                plsc.addupdate_scatter(acc, [si], g)

        @pl.when(jnp.logical_not(last))
        def _():
            pltpu.sync_copy(edges_hbm.at[0:2, pl.ds(base_e, EC)],
                            edgv.at[0:2, pl.ds(0, EC)])

        @pl.when(last)
        def _():
            pltpu.sync_copy(edges_hbm.at[0:2, pl.ds(base_e, ECL)], edgv)

        _zero_vmem(acc1, NP)
        _zero_vmem(acc2, NP)

        @pl.when(jnp.logical_not(last))
        def _():
            deg_pass(EC // L)

        @pl.when(last)
        def _():
            deg_pass(ECL // L)

        pltpu.sync_copy(acc1, shp1.at[wid])
        pltpu.sync_copy(acc2, shp2.at[wid])
        _zero_vmem(acc1, NP)
        plsc.subcore_barrier()

        pull_red(shp1)
        _reduce16(red, lambda j, s: nsl.__setitem__(pl.ds(j * L, L),
                                                    _rsqrt16(s)))
        pull_red(shp2)
        _reduce16(red, lambda j, s: ndl.__setitem__(pl.ds(j * L, L),
                                                    _rsqrt16(s)))

        pltpu.sync_copy(ndl, shf.at[pl.ds(base_n, SL)])
        plsc.subcore_barrier()
        pltpu.sync_copy(shf, full)

        @pl.when(jnp.logical_not(last))
        def _():
            gs_pass(EC // L, acc1)

        @pl.when(last)
        def _():
            gs_pass(ECL // L, acc1)

        pltpu.sync_copy(acc1, shp1.at[wid])
        _zero_vmem(acc2, NP)
        plsc.subcore_barrier()

        pull_red(shp1)

        def wt_out(j, s):
            wv = nsl[pl.ds(j * L, L)] * s
            tsl[pl.ds(j * L, L)] = wv
            ndl[pl.ds(j * L, L)] = ndl[pl.ds(j * L, L)] * wv

        _reduce16(red, wt_out)
        pltpu.sync_copy(tsl, w_out.at[pl.ds(base_n, SL)])
        pltpu.sync_copy(ndl, shf.at[pl.ds(base_n, SL)])
        plsc.subcore_barrier()
        pltpu.sync_copy(shf, full)

        @pl.when(jnp.logical_not(last))
        def _():
            gs_pass(EC // L, acc2)

        @pl.when(last)
        def _():
            gs_pass(ECL // L, acc2)

        pltpu.sync_copy(acc2, shp2.at[wid])
        _zero_vmem(acc1, NP)
        plsc.subcore_barrier()

        pull_red(shp2)
        _reduce16(red, lambda j, s: ndl.__setitem__(
            pl.ds(j * L, L), nsl[pl.ds(j * L, L)] * s))

        @pl.when(jnp.logical_not(last))
        def _():
            pltpu.sync_copy(feats_hbm.at[pl.ds(base_n, SL)], fsl)

        @pl.when(last)
        def _():
            pltpu.sync_copy(feats_hbm.at[pl.ds((NT - 1) * SL, FSL)],
                            fsl.at[pl.ds(0, FSL)])
            izeros = jnp.zeros((L,), jnp.int32)

            @plsc.parallel_loop(0, (SL - FSL) // L, unroll=8)
            def _(i):
                fsl[pl.ds(FSL + i * L, L)] = izeros

        def c_body(j, _):
            fv = fsl[pl.ds(j * L, L)]
            uv = ndl[pl.ds(j * L, L)]
            plsc.addupdate_scatter(acc1, [fv], uv)
            return 0

        lax.fori_loop(0, SL // L, c_body, 0, unroll=4)

        pltpu.sync_copy(acc1, shp1.at[wid])
        plsc.subcore_barrier()

        pull_red(shp1)
        _reduce16(red, lambda j, s: tsl.__setitem__(pl.ds(j * L, L), s))
        pltpu.sync_copy(tsl, c_out.at[pl.ds(base_n, SL)])


_sc_call = functools.partial(
    pl.kernel,
    out_type=[
        jax.ShapeDtypeStruct((NP,), jnp.float32),
        jax.ShapeDtypeStruct((NP,), jnp.float32),
    ],
    mesh=plsc.VectorSubcoreMesh(core_axis_name="c", subcore_axis_name="s"),
    compiler_params=pltpu.CompilerParams(needs_layout_passes=False),
    scratch_types=[
        pltpu.VMEM((2, ECL), jnp.int32),
        pltpu.VMEM((NP,), jnp.float32),
        pltpu.VMEM((NP,), jnp.float32),
        pltpu.VMEM((NP,), jnp.float32),
        pltpu.VMEM((NT, SL), jnp.float32),
        pltpu.VMEM((SL,), jnp.float32),
        pltpu.VMEM((SL,), jnp.float32),
        pltpu.VMEM((SL,), jnp.float32),
        pltpu.VMEM((SL,), jnp.int32),
        pltpu.VMEM_SHARED((NT, NP), jnp.float32),
        pltpu.VMEM_SHARED((NT, NP), jnp.float32),
        pltpu.VMEM_SHARED((NP,), jnp.float32),
        pltpu.SemaphoreType.DMA,
    ],
)(_sc_body)


def _tc_body(c_ref, w_ref, emb_ref, w0_ref, b0_ref, w1_ref, b1_ref,
             wreg_ref, o_ref):
    w0 = w0_ref[...].astype(jnp.bfloat16).astype(jnp.float32)
    w1 = w1_ref[...].astype(jnp.bfloat16).astype(jnp.float32)
    ct = c_ref[0:VOCAB, :]
    z = jnp.sum(ct * emb_ref[...], axis=0, keepdims=True)
    sw = jnp.sum(w_ref[...])
    r1 = jnp.sum(z.reshape(H, 1) * w0, axis=0, keepdims=True)
    r1 = r1 + sw * b0_ref[...]
    r2 = jnp.sum(r1.reshape(H, 1) * w1, axis=0, keepdims=True)
    r2 = r2 + float(N) * b1_ref[...]
    o_ref[...] = jnp.sum(r2 * wreg_ref[...], axis=1, keepdims=True)


def kernel(feats, edge_index, emb, W0, b0, W1, b1, Wreg):
    c, w = _sc_call(edge_index.astype(jnp.int32), feats.astype(jnp.int32))

    out = pl.pallas_call(
        _tc_body,
        out_shape=jax.ShapeDtypeStruct((1, 1), jnp.float32),
    )(c.reshape(NP, 1), w.reshape(1, NP), emb,
      W0, b0.reshape(1, H), W1, b1.reshape(1, H), Wreg)
    return out


# speedup vs baseline: 51.9808x; 1.0659x over previous
import functools

import jax
import jax.numpy as jnp
from jax import lax
from jax.experimental import pallas as pl
from jax.experimental.pallas import tpu as pltpu
from jax.experimental.pallas import tpu_sc as plsc

N = 10000
E = 320000
H = 128
VOCAB = 10000

L = 16
NT = 16
NP = 10240
SL = NP // NT
EC = 19968
ECL = E - (NT - 1) * EC
FSL = N - 15 * SL


def _rsqrt16(x):
    xi = plsc.bitcast(x, jnp.int32)
    yi = jnp.int32(0x5F3759DF) - lax.shift_right_arithmetic(xi, 1)
    y = plsc.bitcast(yi, jnp.float32)
    for _ in range(3):
        y = y * (1.5 - 0.5 * x * y * y)
    return jnp.where(x > 0.0, y, 0.0)


def _zero_vmem(ref, n):
    zeros = jnp.zeros((L,), jnp.float32)

    @plsc.parallel_loop(0, n // L, unroll=8)
    def _(i):
        ref[pl.ds(i * L, L)] = zeros


def _reduce16(red, out_slice_fn):
    @plsc.parallel_loop(0, SL // L, unroll=2)
    def _(j):
        s = red[0, pl.ds(j * L, L)]
        for t in range(1, NT):
            s = s + red[t, pl.ds(j * L, L)]
        out_slice_fn(j, s)


def _sc_body(edges_hbm, feats_hbm, c_out, w_out,
             edgv, acc1, acc2, full, red, red2, nsl, ndl, tsl, fsl,
             shp1, shp2, shf, sem):
    core = lax.axis_index("c")
    sid = lax.axis_index("s")

    @pl.when(core == 0)
    def _():
        wid = sid
        base_e = wid * EC
        base_n = wid * SL
        ones = jnp.ones((L,), jnp.float32)
        last = sid == NT - 1

        def pull_red(shp):
            cps = [pltpu.async_copy(shp.at[t, pl.ds(base_n, SL)], red.at[t],
                                    sem) for t in range(NT)]
            for cp in cps:
                cp.wait()

        def deg_pass(nv):
            @plsc.parallel_loop(0, nv, unroll=4)
            def _(i):
                si = edgv[0, pl.ds(i * L, L)]
                di = edgv[1, pl.ds(i * L, L)]
                plsc.addupdate_scatter(acc1, [si], ones)
                plsc.addupdate_scatter(acc2, [di], ones)

        def gs_pass(nv, acc):
            @plsc.parallel_loop(0, nv, unroll=4)
            def _(i):
                si = edgv[0, pl.ds(i * L, L)]
                di = edgv[1, pl.ds(i * L, L)]
                g = plsc.load_gather(full, [di])
                plsc.addupdate_scatter(acc, [si], g)

        ecp = pltpu.async_copy(edges_hbm.at[0:2, pl.ds(base_e, EC)],
                               edgv.at[0:2, pl.ds(0, EC)], sem)

        _zero_vmem(acc1, NP)
        _zero_vmem(acc2, NP)
        ecp.wait()

        @pl.when(last)
        def _():
            pltpu.sync_copy(edges_hbm.at[0:2, pl.ds(base_e + EC, ECL - EC)],
                            edgv.at[0:2, pl.ds(EC, ECL - EC)])

        @pl.when(jnp.logical_not(last))
        def _():
            deg_pass(EC // L)

        @pl.when(last)
        def _():
            deg_pass(ECL // L)

        pltpu.sync_copy(acc1, shp1.at[wid])
        pltpu.sync_copy(acc2, shp2.at[wid])
        _zero_vmem(acc1, NP)
        plsc.subcore_barrier()

        cps = ([pltpu.async_copy(shp1.at[t, pl.ds(base_n, SL)], red.at[t],
                                 sem) for t in range(NT)] +
               [pltpu.async_copy(shp2.at[t, pl.ds(base_n, SL)], red2.at[t],
                                 sem) for t in range(NT)])
        for cp in cps:
            cp.wait()
        _reduce16(red, lambda j, s: nsl.__setitem__(pl.ds(j * L, L),
                                                    _rsqrt16(s)))
        _reduce16(red2, lambda j, s: ndl.__setitem__(pl.ds(j * L, L),
                                                     _rsqrt16(s)))

        pltpu.sync_copy(ndl, shf.at[pl.ds(base_n, SL)])
        plsc.subcore_barrier()
        pltpu.sync_copy(shf, full)

        @pl.when(jnp.logical_not(last))
        def _():
            gs_pass(EC // L, acc1)

        @pl.when(last)
        def _():
            gs_pass(ECL // L, acc1)

        pltpu.sync_copy(acc1, shp1.at[wid])
        _zero_vmem(acc2, NP)
        plsc.subcore_barrier()

        pull_red(shp1)

        def wt_out(j, s):
            wv = nsl[pl.ds(j * L, L)] * s
            tsl[pl.ds(j * L, L)] = wv
            ndl[pl.ds(j * L, L)] = ndl[pl.ds(j * L, L)] * wv

        _reduce16(red, wt_out)
        pltpu.sync_copy(tsl, w_out.at[pl.ds(base_n, SL)])
        pltpu.sync_copy(ndl, shf.at[pl.ds(base_n, SL)])
        plsc.subcore_barrier()
        pltpu.sync_copy(shf, full)

        @pl.when(jnp.logical_not(last))
        def _():
            gs_pass(EC // L, acc2)

        @pl.when(last)
        def _():
            gs_pass(ECL // L, acc2)

        pltpu.sync_copy(acc2, shp2.at[wid])
        _zero_vmem(acc1, NP)
        plsc.subcore_barrier()

        pull_red(shp2)
        _reduce16(red, lambda j, s: ndl.__setitem__(
            pl.ds(j * L, L), nsl[pl.ds(j * L, L)] * s))

        @pl.when(jnp.logical_not(last))
        def _():
            pltpu.sync_copy(feats_hbm.at[pl.ds(base_n, SL)], fsl)

        @pl.when(last)
        def _():
            pltpu.sync_copy(feats_hbm.at[pl.ds((NT - 1) * SL, FSL)],
                            fsl.at[pl.ds(0, FSL)])
            izeros = jnp.zeros((L,), jnp.int32)

            @plsc.parallel_loop(0, (SL - FSL) // L, unroll=8)
            def _(i):
                fsl[pl.ds(FSL + i * L, L)] = izeros

        def c_body(j, _):
            fv = fsl[pl.ds(j * L, L)]
            uv = ndl[pl.ds(j * L, L)]
            plsc.addupdate_scatter(acc1, [fv], uv)
            return 0

        lax.fori_loop(0, SL // L, c_body, 0, unroll=4)

        pltpu.sync_copy(acc1, shp1.at[wid])
        plsc.subcore_barrier()

        pull_red(shp1)
        _reduce16(red, lambda j, s: tsl.__setitem__(pl.ds(j * L, L), s))
        pltpu.sync_copy(tsl, c_out.at[pl.ds(base_n, SL)])


_sc_call = functools.partial(
    pl.kernel,
    out_type=[
        jax.ShapeDtypeStruct((NP,), jnp.float32),
        jax.ShapeDtypeStruct((NP,), jnp.float32),
    ],
    mesh=plsc.VectorSubcoreMesh(core_axis_name="c", subcore_axis_name="s"),
    compiler_params=pltpu.CompilerParams(needs_layout_passes=False),
    scratch_types=[
        pltpu.VMEM((2, ECL), jnp.int32),
        pltpu.VMEM((NP,), jnp.float32),
        pltpu.VMEM((NP,), jnp.float32),
        pltpu.VMEM((NP,), jnp.float32),
        pltpu.VMEM((NT, SL), jnp.float32),
        pltpu.VMEM((NT, SL), jnp.float32),
        pltpu.VMEM((SL,), jnp.float32),
        pltpu.VMEM((SL,), jnp.float32),
        pltpu.VMEM((SL,), jnp.float32),
        pltpu.VMEM((SL,), jnp.int32),
        pltpu.VMEM_SHARED((NT, NP), jnp.float32),
        pltpu.VMEM_SHARED((NT, NP), jnp.float32),
        pltpu.VMEM_SHARED((NP,), jnp.float32),
        pltpu.SemaphoreType.DMA,
    ],
)(_sc_body)


def _tc_body(c_ref, w_ref, emb_ref, w0_ref, b0_ref, w1_ref, b1_ref,
             wreg_ref, o_ref):
    w0 = w0_ref[...].astype(jnp.bfloat16).astype(jnp.float32)
    w1 = w1_ref[...].astype(jnp.bfloat16).astype(jnp.float32)
    ct = c_ref[0:VOCAB, :]
    z = jnp.sum(ct * emb_ref[...], axis=0, keepdims=True)
    sw = jnp.sum(w_ref[...])
    r1 = jnp.sum(z.reshape(H, 1) * w0, axis=0, keepdims=True)
    r1 = r1 + sw * b0_ref[...]
    r2 = jnp.sum(r1.reshape(H, 1) * w1, axis=0, keepdims=True)
    r2 = r2 + float(N) * b1_ref[...]
    o_ref[...] = jnp.sum(r2 * wreg_ref[...], axis=1, keepdims=True)


def kernel(feats, edge_index, emb, W0, b0, W1, b1, Wreg):
    c, w = _sc_call(edge_index.astype(jnp.int32), feats.astype(jnp.int32))

    out = pl.pallas_call(
        _tc_body,
        out_shape=jax.ShapeDtypeStruct((1, 1), jnp.float32),
    )(c.reshape(NP, 1), w.reshape(1, NP), emb,
      W0, b0.reshape(1, H), W1, b1.reshape(1, H), Wreg)
    return out
